# speedup vs baseline: 5.3749x; 5.3749x over previous
import functools

import jax
import jax.numpy as jnp
from jax.experimental import pallas as pl
from jax.experimental.pallas import tpu as pltpu
from jax.experimental.pallas import tpu_sc as plsc

N = 10000
E = 320000
D = 128

NC = 2
NS = 16
L = 16

NPAD = 10240
ROWS_PER_SUB = NPAD // NS
K = 80
EPC = E // NC
EPS = EPC // NS
NITER = EPS // K


def _sc_aggregate(h, src, dst, zeros2d, zeros1d, ones1d, with_deg):
  out_type = [jax.ShapeDtypeStruct((NC * NPAD, D), jnp.float32)]
  if with_deg:
    out_type.append(jax.ShapeDtypeStruct((NC * NPAD,), jnp.float32))

  scratch = [
      pltpu.VMEM((K,), jnp.int32),
      pltpu.VMEM((K,), jnp.int32),
      pltpu.VMEM((K, D), jnp.float32),
      pltpu.VMEM((K,), jnp.float32),
      pltpu.VMEM_SHARED((NPAD, D), jnp.float32),
      pltpu.VMEM_SHARED((NPAD,), jnp.float32),
      pltpu.SemaphoreType.DMA,
  ]

  mesh = plsc.VectorSubcoreMesh(core_axis_name="c", subcore_axis_name="s")

  def body(h_hbm, src_hbm, dst_hbm, z2_hbm, z1_hbm, ones_hbm, *rest):
    if with_deg:
      acc_out, deg_out, src_v, dst_v, rows_v, ones_v, acc_sh, deg_sh, sem = rest
    else:
      acc_out, src_v, dst_v, rows_v, ones_v, acc_sh, deg_sh, sem = rest
      deg_out = None
    c = jax.lax.axis_index("c")
    s = jax.lax.axis_index("s")
    rbase = s * ROWS_PER_SUB

    pltpu.sync_copy(z2_hbm, acc_sh.at[pl.ds(rbase, ROWS_PER_SUB)])
    if with_deg:
      pltpu.sync_copy(z1_hbm, deg_sh.at[pl.ds(rbase, ROWS_PER_SUB)])
      pltpu.sync_copy(ones_hbm, ones_v)
    plsc.subcore_barrier()

    ebase = c * EPC + s * EPS

    def step(i, carry):
      off = ebase + i * K
      pltpu.sync_copy(src_hbm.at[pl.ds(off, K)], src_v)
      pltpu.sync_copy(dst_hbm.at[pl.ds(off, K)], dst_v)
      pltpu.async_copy(h_hbm.at[src_v], rows_v, sem).wait()
      pltpu.sync_copy(rows_v, acc_sh.at[dst_v], add=True)
      if with_deg:
        pltpu.sync_copy(ones_v, deg_sh.at[dst_v], add=True)
      return carry

    jax.lax.fori_loop(0, NITER, step, 0)
    plsc.subcore_barrier()

    obase = c * NPAD + rbase
    pltpu.sync_copy(acc_sh.at[pl.ds(rbase, ROWS_PER_SUB)],
                    acc_out.at[pl.ds(obase, ROWS_PER_SUB)])
    if with_deg:
      pltpu.sync_copy(deg_sh.at[pl.ds(rbase, ROWS_PER_SUB)],
                      deg_out.at[pl.ds(obase, ROWS_PER_SUB)])

  fn = pl.kernel(body, out_type=out_type, mesh=mesh, scratch_types=scratch,
                 name="sc_gather_scatter_add")
  res = fn(h, src, dst, zeros2d, zeros1d, ones1d)
  if with_deg:
    return res[0], res[1]
  return res[0], None


def _tc_layer1(x, acc, degp, Ws, Wn, b):
  def body(x_ref, acc_ref, degp_ref, Ws_ref, Wn_ref, b_ref, h_ref, inv_ref):
    deg = degp_ref[0, :N, :] + degp_ref[1, :N, :]
    invdeg = 1.0 / jnp.maximum(deg, 1.0)
    agg = (acc_ref[0, :N, :] + acc_ref[1, :N, :]) * invdeg
    z = (jnp.dot(x_ref[...], Ws_ref[...], preferred_element_type=jnp.float32)
         + jnp.dot(agg, Wn_ref[...], preferred_element_type=jnp.float32)
         + b_ref[...])
    h_ref[...] = jnp.maximum(z, 0.0)
    inv_ref[...] = invdeg

  return pl.pallas_call(
      body,
      out_shape=[jax.ShapeDtypeStruct((N, D), jnp.float32),
                 jax.ShapeDtypeStruct((N, 1), jnp.float32)],
  )(x, acc.reshape(NC, NPAD, D), degp.reshape(NC, NPAD, 1), Ws, Wn,
    b.reshape(1, -1))


def _tc_layer2_head(h1, acc, invdeg, Ws, Wn, b, Wc1, bc1, Wc2, bc2):
  def body(h1_ref, acc_ref, inv_ref, Ws_ref, Wn_ref, b_ref, Wc1_ref,
           bc1_ref, Wc2_ref, bc2_ref, out_ref):
    agg = (acc_ref[0, :N, :] + acc_ref[1, :N, :]) * inv_ref[...]
    h1v = h1_ref[...]
    z = (jnp.dot(h1v, Ws_ref[...], preferred_element_type=jnp.float32)
         + jnp.dot(agg, Wn_ref[...], preferred_element_type=jnp.float32)
         + b_ref[...])
    h2 = jnp.maximum(z, 0.0)
    hc = (jnp.dot(h1v, Wc1_ref[:D, :], preferred_element_type=jnp.float32)
          + jnp.dot(h2, Wc1_ref[D:, :], preferred_element_type=jnp.float32)
          + bc1_ref[...])
    hc = jnp.maximum(hc, 0.0)
    out_ref[...] = (jnp.dot(hc, Wc2_ref[...],
                            preferred_element_type=jnp.float32)
                    + bc2_ref[...])

  C = bc2.shape[0]
  return pl.pallas_call(
      body,
      out_shape=jax.ShapeDtypeStruct((N, C), jnp.float32),
  )(h1, acc.reshape(NC, NPAD, D), invdeg, Ws, Wn, b.reshape(1, -1),
    Wc1, bc1.reshape(1, -1), Wc2, bc2.reshape(1, -1))


def kernel(x, edge_index, Ws1, Wn1, b1, Ws2, Wn2, b2, Wc1, bc1, Wc2, bc2):
  src = edge_index[0]
  dst = edge_index[1]
  zeros2d = jnp.zeros((ROWS_PER_SUB, D), jnp.float32)
  zeros1d = jnp.zeros((ROWS_PER_SUB,), jnp.float32)
  ones1d = jnp.ones((K,), jnp.float32)

  acc1, degp = _sc_aggregate(x, src, dst, zeros2d, zeros1d, ones1d,
                             with_deg=True)
  h1, invdeg = _tc_layer1(x, acc1, degp, Ws1, Wn1, b1)
  acc2, _ = _sc_aggregate(h1, src, dst, zeros2d, zeros1d, ones1d,
                          with_deg=False)
  return _tc_layer2_head(h1, acc2, invdeg, Ws2, Wn2, b2, Wc1, bc1, Wc2, bc2)


# speedup vs baseline: 11.6868x; 2.1743x over previous
import functools

import jax
import jax.numpy as jnp
from jax.experimental import pallas as pl
from jax.experimental.pallas import tpu as pltpu
from jax.experimental.pallas import tpu_sc as plsc

N = 10000
E = 320000
D = 128

NC = 2
NS = 16
L = 16

NPAD = 10112
ROWS_PER_SUB = NPAD // NS
K = 80
EPC = E // NC
EPS = EPC // NS
NITER = EPS // K


def _sc_aggregate(h, src2, dst2, zeros2d, zeros1d, ones1d, with_deg):
  out_type = [jax.ShapeDtypeStruct((NC * NPAD, D), jnp.float32)]
  if with_deg:
    out_type.append(jax.ShapeDtypeStruct((NC * NPAD,), jnp.float32))

  scratch = [
      pltpu.VMEM((EPS,), jnp.int32),
      pltpu.VMEM((NITER, K), jnp.int32),
      pltpu.VMEM((2, K, D), jnp.float32),
      pltpu.VMEM((K,), jnp.float32),
      pltpu.VMEM_SHARED((NPAD, D), jnp.float32),
      pltpu.VMEM_SHARED((NPAD,), jnp.float32),
      pltpu.SemaphoreType.DMA,
      pltpu.SemaphoreType.DMA,
  ]

  mesh = plsc.VectorSubcoreMesh(core_axis_name="c", subcore_axis_name="s")

  def body(h_hbm, src_hbm, dst_hbm, z2_hbm, z1_hbm, ones_hbm, *rest):
    if with_deg:
      (acc_out, deg_out, src_v, dst_v, rows_v, ones_v, acc_sh, deg_sh,
       sem0, sem1) = rest
    else:
      acc_out, src_v, dst_v, rows_v, ones_v, acc_sh, deg_sh, sem0, sem1 = rest
      deg_out = None
    c = jax.lax.axis_index("c")
    s = jax.lax.axis_index("s")
    w = c * NS + s
    rbase = s * ROWS_PER_SUB

    pltpu.sync_copy(z2_hbm, acc_sh.at[pl.ds(rbase, ROWS_PER_SUB)])
    pltpu.sync_copy(src_hbm.at[w], src_v)
    pltpu.sync_copy(dst_hbm.at[w], dst_v)
    if with_deg:
      @pl.when(s == 0)
      def _():
        pltpu.sync_copy(z1_hbm, deg_sh)
      pltpu.sync_copy(ones_hbm, ones_v)
    plsc.subcore_barrier()

    sems = (sem0, sem1)

    def gather(j, b):
      off = pl.multiple_of(j * K, 8)
      pltpu.async_copy(h_hbm.at[src_v.at[pl.ds(off, K)]], rows_v.at[b],
                       sems[b])

    def wait_gather(j, b):
      off = pl.multiple_of(j * K, 8)
      pltpu.make_async_copy(h_hbm.at[src_v.at[pl.ds(off, K)]], rows_v.at[b],
                            sems[b]).wait()

    def scatter(j, b):
      pltpu.sync_copy(rows_v.at[b], acc_sh.at[dst_v.at[j]], add=True)
      if with_deg:
        pltpu.sync_copy(ones_v, deg_sh.at[dst_v.at[j]], add=True)

    gather(0, 0)

    def pair(t, carry):
      j0 = 2 * t
      gather(j0 + 1, 1)
      wait_gather(j0, 0)
      scatter(j0, 0)
      gather(j0 + 2, 0)
      wait_gather(j0 + 1, 1)
      scatter(j0 + 1, 1)
      return carry

    jax.lax.fori_loop(0, (NITER - 1) // 2, pair, 0)
    wait_gather(NITER - 1, 0)
    scatter(NITER - 1, 0)
    plsc.subcore_barrier()

    obase = c * NPAD + rbase
    pltpu.sync_copy(acc_sh.at[pl.ds(rbase, ROWS_PER_SUB)],
                    acc_out.at[pl.ds(obase, ROWS_PER_SUB)])
    if with_deg:
      @pl.when(s == 0)
      def _():
        pltpu.sync_copy(deg_sh, deg_out.at[pl.ds(c * NPAD, NPAD)])

  fn = pl.kernel(body, out_type=out_type, mesh=mesh, scratch_types=scratch,
                 name="sc_gather_scatter_add")
  res = fn(h, src2, dst2, zeros2d, zeros1d, ones1d)
  if with_deg:
    return res[0], res[1]
  return res[0], None


def _tc_layer1(x, acc, degp, Ws, Wn, b):
  def body(x_ref, acc_ref, degp_ref, Ws_ref, Wn_ref, b_ref, h_ref, inv_ref):
    deg = degp_ref[0, :N, :] + degp_ref[1, :N, :]
    invdeg = 1.0 / jnp.maximum(deg, 1.0)
    agg = (acc_ref[0, :N, :] + acc_ref[1, :N, :]) * invdeg
    z = (jnp.dot(x_ref[...], Ws_ref[...], preferred_element_type=jnp.float32)
         + jnp.dot(agg, Wn_ref[...], preferred_element_type=jnp.float32)
         + b_ref[...])
    h_ref[...] = jnp.maximum(z, 0.0)
    inv_ref[...] = invdeg

  return pl.pallas_call(
      body,
      out_shape=[jax.ShapeDtypeStruct((N, D), jnp.float32),
                 jax.ShapeDtypeStruct((N, 1), jnp.float32)],
  )(x, acc.reshape(NC, NPAD, D), degp.reshape(NC, NPAD, 1), Ws, Wn,
    b.reshape(1, -1))


def _tc_layer2_head(h1, acc, invdeg, Ws, Wn, b, Wc1, bc1, Wc2, bc2):
  def body(h1_ref, acc_ref, inv_ref, Ws_ref, Wn_ref, b_ref, Wc1_ref,
           bc1_ref, Wc2_ref, bc2_ref, out_ref):
    agg = (acc_ref[0, :N, :] + acc_ref[1, :N, :]) * inv_ref[...]
    h1v = h1_ref[...]
    z = (jnp.dot(h1v, Ws_ref[...], preferred_element_type=jnp.float32)
         + jnp.dot(agg, Wn_ref[...], preferred_element_type=jnp.float32)
         + b_ref[...])
    h2 = jnp.maximum(z, 0.0)
    hc = (jnp.dot(h1v, Wc1_ref[:D, :], preferred_element_type=jnp.float32)
          + jnp.dot(h2, Wc1_ref[D:, :], preferred_element_type=jnp.float32)
          + bc1_ref[...])
    hc = jnp.maximum(hc, 0.0)
    out_ref[...] = (jnp.dot(hc, Wc2_ref[...],
                            preferred_element_type=jnp.float32)
                    + bc2_ref[...])

  C = bc2.shape[0]
  return pl.pallas_call(
      body,
      out_shape=jax.ShapeDtypeStruct((N, C), jnp.float32),
  )(h1, acc.reshape(NC, NPAD, D), invdeg, Ws, Wn, b.reshape(1, -1),
    Wc1, bc1.reshape(1, -1), Wc2, bc2.reshape(1, -1))


def kernel(x, edge_index, Ws1, Wn1, b1, Ws2, Wn2, b2, Wc1, bc1, Wc2, bc2):
  src = edge_index[0].reshape(NC * NS, EPS)
  dst = edge_index[1].reshape(NC * NS, NITER, K)
  zeros2d = jnp.zeros((ROWS_PER_SUB, D), jnp.float32)
  zeros1d = jnp.zeros((NPAD,), jnp.float32)
  ones1d = jnp.ones((K,), jnp.float32)

  acc1, degp = _sc_aggregate(x, src, dst, zeros2d, zeros1d, ones1d,
                             with_deg=True)
  h1, invdeg = _tc_layer1(x, acc1, degp, Ws1, Wn1, b1)
  acc2, _ = _sc_aggregate(h1, src, dst, zeros2d, zeros1d, ones1d,
                          with_deg=False)
  return _tc_layer2_head(h1, acc2, invdeg, Ws2, Wn2, b2, Wc1, bc1, Wc2, bc2)


# speedup vs baseline: 12.6933x; 1.0861x over previous
import functools

import jax
import jax.numpy as jnp
from jax.experimental import pallas as pl
from jax.experimental.pallas import tpu as pltpu
from jax.experimental.pallas import tpu_sc as plsc

N = 10000
E = 320000
D = 128

NC = 2
NS = 16
L = 16

NPAD = 10112
ROWS_PER_SUB = NPAD // NS
K = 128
EPS = E // (NC * NS)
EPSP = 10240
GC = 8
NITER = EPSP // K
NG = NITER // GC


def _sc_aggregate(h, srcp, dstp, zeros2d, zeros1d, ones1d, with_deg):
  out_type = [jax.ShapeDtypeStruct((NC * NPAD, D), jnp.float32)]
  if with_deg:
    out_type.append(jax.ShapeDtypeStruct((NC * NPAD,), jnp.float32))

  scratch = [
      pltpu.VMEM((2 * GC * K,), jnp.int32),
      pltpu.VMEM((2, GC, K), jnp.int32),
      pltpu.VMEM((2, K, D), jnp.float32),
      pltpu.VMEM((K,), jnp.float32),
      pltpu.VMEM_SHARED((NPAD, D), jnp.float32),
      pltpu.VMEM_SHARED((NPAD,), jnp.float32),
      pltpu.SemaphoreType.DMA,
      pltpu.SemaphoreType.DMA,
      pltpu.SemaphoreType.DMA,
      pltpu.SemaphoreType.DMA,
  ]

  mesh = plsc.VectorSubcoreMesh(core_axis_name="c", subcore_axis_name="s")

  def body(h_hbm, src_hbm, dst_hbm, z2_hbm, z1_hbm, ones_hbm, *rest):
    if with_deg:
      (acc_out, deg_out, src_v, dst_v, rows_v, ones_v, acc_sh, deg_sh,
       sem_g0, sem_g1, sem_i, sem_d) = rest
    else:
      (acc_out, src_v, dst_v, rows_v, ones_v, acc_sh, deg_sh,
       sem_g0, sem_g1, sem_i, sem_d) = rest
      deg_out = None
    c = jax.lax.axis_index("c")
    s = jax.lax.axis_index("s")
    w = c * NS + s
    rbase = s * ROWS_PER_SUB
    gsems = (sem_g0, sem_g1)

    pltpu.sync_copy(z2_hbm, acc_sh.at[pl.ds(rbase, ROWS_PER_SUB)])
    pltpu.sync_copy(src_hbm.at[w, pl.ds(0, GC * K)],
                    src_v.at[pl.ds(0, GC * K)])
    pltpu.sync_copy(dst_hbm.at[w, 0], dst_v.at[0])
    if with_deg:
      @pl.when(s == 0)
      def _():
        pltpu.sync_copy(z1_hbm, deg_sh)
      pltpu.sync_copy(ones_hbm, ones_v)
    plsc.subcore_barrier()

    def src_slice(half, i):
      off = pl.multiple_of(half * (GC * K) + i * K, 8)
      return src_v.at[pl.ds(off, K)]

    def issue_gather(half, i, b):
      pltpu.async_copy(h_hbm.at[src_slice(half, i)], rows_v.at[b], gsems[b])

    def wait_gather(half, i, b):
      pltpu.make_async_copy(h_hbm.at[src_slice(half, i)], rows_v.at[b],
                            gsems[b]).wait()

    def wait_deg():
      pltpu.make_async_copy(ones_v, deg_sh.at[dst_v.at[0, 0]], sem_d).wait()

    issue_gather(0, 0, 0)
    issue_gather(0, 1, 1)

    def group(g, carry):
      half = jax.lax.rem(g, 2)
      nhalf = 1 - half

      @pl.when(g < NG - 1)
      def _():
        soff = pl.multiple_of((g + 1) * GC * K, 8)
        pltpu.async_copy(src_hbm.at[w, pl.ds(soff, GC * K)],
                         src_v.at[pl.ds(nhalf * (GC * K), GC * K)], sem_i)
        pltpu.async_copy(dst_hbm.at[w, g + 1], dst_v.at[nhalf], sem_i)

      for i in range(GC):
        b = i % 2
        wait_gather(half, i, b)
        pltpu.sync_copy(rows_v.at[b], acc_sh.at[dst_v.at[half, i]], add=True)
        if with_deg:
          if i == 0:
            @pl.when(g > 0)
            def _():
              wait_deg()
          else:
            wait_deg()
          pltpu.async_copy(ones_v, deg_sh.at[dst_v.at[half, i]], sem_d,
                           add=True)
        if i < GC - 2:
          issue_gather(half, i + 2, b)
        else:
          @pl.when(g < NG - 1)
          def _():
            if i == GC - 2:
              pltpu.make_async_copy(
                  src_hbm.at[w, pl.ds(pl.multiple_of((g + 1) * GC * K, 8),
                                      GC * K)],
                  src_v.at[pl.ds(nhalf * (GC * K), GC * K)], sem_i).wait()
              pltpu.make_async_copy(dst_hbm.at[w, g + 1], dst_v.at[nhalf],
                                    sem_i).wait()
            issue_gather(nhalf, i - (GC - 2), b)
      return carry

    jax.lax.fori_loop(0, NG, group, 0)
    if with_deg:
      wait_deg()
    plsc.subcore_barrier()

    obase = c * NPAD + rbase
    pltpu.sync_copy(acc_sh.at[pl.ds(rbase, ROWS_PER_SUB)],
                    acc_out.at[pl.ds(obase, ROWS_PER_SUB)])
    if with_deg:
      @pl.when(s == 0)
      def _():
        pltpu.sync_copy(deg_sh, deg_out.at[pl.ds(c * NPAD, NPAD)])

  fn = pl.kernel(body, out_type=out_type, mesh=mesh, scratch_types=scratch,
                 name="sc_gather_scatter_add")
  res = fn(h, srcp, dstp, zeros2d, zeros1d, ones1d)
  if with_deg:
    return res[0], res[1]
  return res[0], None


def _tc_layer1(x, acc, degp, Ws, Wn, b):
  def body(x_ref, acc_ref, degp_ref, Ws_ref, Wn_ref, b_ref, h_ref, inv_ref):
    deg = degp_ref[0, :N, :] + degp_ref[1, :N, :]
    invdeg = 1.0 / jnp.maximum(deg, 1.0)
    agg = (acc_ref[0, :N, :] + acc_ref[1, :N, :]) * invdeg
    z = (jnp.dot(x_ref[...], Ws_ref[...], preferred_element_type=jnp.float32)
         + jnp.dot(agg, Wn_ref[...], preferred_element_type=jnp.float32)
         + b_ref[...])
    h_ref[...] = jnp.maximum(z, 0.0)
    inv_ref[...] = invdeg

  return pl.pallas_call(
      body,
      out_shape=[jax.ShapeDtypeStruct((N, D), jnp.float32),
                 jax.ShapeDtypeStruct((N, 1), jnp.float32)],
  )(x, acc.reshape(NC, NPAD, D), degp.reshape(NC, NPAD, 1), Ws, Wn,
    b.reshape(1, -1))


def _tc_layer2_head(h1, acc, invdeg, Ws, Wn, b, Wc1, bc1, Wc2, bc2):
  def body(h1_ref, acc_ref, inv_ref, Ws_ref, Wn_ref, b_ref, Wc1_ref,
           bc1_ref, Wc2_ref, bc2_ref, out_ref):
    agg = (acc_ref[0, :N, :] + acc_ref[1, :N, :]) * inv_ref[...]
    h1v = h1_ref[...]
    z = (jnp.dot(h1v, Ws_ref[...], preferred_element_type=jnp.float32)
         + jnp.dot(agg, Wn_ref[...], preferred_element_type=jnp.float32)
         + b_ref[...])
    h2 = jnp.maximum(z, 0.0)
    hc = (jnp.dot(h1v, Wc1_ref[:D, :], preferred_element_type=jnp.float32)
          + jnp.dot(h2, Wc1_ref[D:, :], preferred_element_type=jnp.float32)
          + bc1_ref[...])
    hc = jnp.maximum(hc, 0.0)
    out_ref[...] = (jnp.dot(hc, Wc2_ref[...],
                            preferred_element_type=jnp.float32)
                    + bc2_ref[...])

  C = bc2.shape[0]
  return pl.pallas_call(
      body,
      out_shape=jax.ShapeDtypeStruct((N, C), jnp.float32),
  )(h1, acc.reshape(NC, NPAD, D), invdeg, Ws, Wn, b.reshape(1, -1),
    Wc1, bc1.reshape(1, -1), Wc2, bc2.reshape(1, -1))


def kernel(x, edge_index, Ws1, Wn1, b1, Ws2, Wn2, b2, Wc1, bc1, Wc2, bc2):
  NW = NC * NS
  npad_e = EPSP - EPS
  pad_src = (jnp.arange(NW * npad_e, dtype=jnp.int32) % N).reshape(NW, npad_e)
  pad_dst = (N + jnp.arange(NW * npad_e, dtype=jnp.int32) % (NPAD - N)
             ).astype(jnp.int32).reshape(NW, npad_e)
  src = jnp.concatenate([edge_index[0].reshape(NW, EPS), pad_src], axis=1)
  dst = jnp.concatenate([edge_index[1].reshape(NW, EPS), pad_dst],
                        axis=1).reshape(NW, NG, GC, K)
  zeros2d = jnp.zeros((ROWS_PER_SUB, D), jnp.float32)
  zeros1d = jnp.zeros((NPAD,), jnp.float32)
  ones1d = jnp.ones((K,), jnp.float32)

  acc1, degp = _sc_aggregate(x, src, dst, zeros2d, zeros1d, ones1d,
                             with_deg=True)
  h1, invdeg = _tc_layer1(x, acc1, degp, Ws1, Wn1, b1)
  acc2, _ = _sc_aggregate(h1, src, dst, zeros2d, zeros1d, ones1d,
                          with_deg=False)
  return _tc_layer2_head(h1, acc2, invdeg, Ws2, Wn2, b2, Wc1, bc1, Wc2, bc2)
